# Initial kernel scaffold; baseline (speedup 1.0000x reference)
import jax
import jax.numpy as jnp
from jax.experimental import pallas as pl


def kernel(x, edge_index, edge_attr, We1, be1, We2, be2, We3, be3, Wn11, bn11, Wn12, bn12, Wn13, bn13, Wn21, bn21, Wn22, bn22, Wn23, bn23):
    raise NotImplementedError("write your pallas kernel here")


# speedup vs baseline: 2.4601x; 2.4601x over previous
import functools

import jax
import jax.numpy as jnp
from jax import lax
from jax.experimental import pallas as pl
from jax.experimental.pallas import tpu as pltpu
from jax.experimental.pallas import tpu_sc as plsc

NC = 2
NS = 16
NW = NC * NS
CH = 128
D2 = 144

_N_PAD = 10240


def _gather_make(n, e, d):
    n_chunks = e // CH
    base_c, rem_c = n_chunks // NW, n_chunks % NW
    mesh = plsc.VectorSubcoreMesh(core_axis_name="c", subcore_axis_name="s")

    @functools.partial(
        pl.kernel,
        out_type=jax.ShapeDtypeStruct((2, e, d), jnp.float32),
        mesh=mesh,
        scratch_types=[
            pltpu.VMEM((CH,), jnp.int32),
            pltpu.VMEM((CH,), jnp.int32),
            pltpu.VMEM((CH, d), jnp.float32),
            pltpu.VMEM((CH, d), jnp.float32),
            pltpu.SemaphoreType.DMA,
            pltpu.SemaphoreType.DMA,
        ],
    )
    def k(x_hbm, row_hbm, col_hbm, out_hbm, idx_r, idx_c, buf_r, buf_c, sem_r, sem_c):
        cid = lax.axis_index("c")
        sid = lax.axis_index("s")
        wid = sid * NC + cid
        n_my = base_c + jnp.where(wid < rem_c, 1, 0)

        def body(i, _):
            off = (wid + i * NW) * CH
            pltpu.sync_copy(row_hbm.at[pl.ds(off, CH)], idx_r)
            pltpu.sync_copy(col_hbm.at[pl.ds(off, CH)], idx_c)
            cp_r = pltpu.async_copy(x_hbm.at[idx_r], buf_r, sem_r)
            cp_c = pltpu.async_copy(x_hbm.at[idx_c], buf_c, sem_c)
            cp_r.wait()
            cp_c.wait()
            pltpu.sync_copy(buf_r, out_hbm.at[0, pl.ds(off, CH)])
            pltpu.sync_copy(buf_c, out_hbm.at[1, pl.ds(off, CH)])
            return 0

        lax.fori_loop(0, n_my, body, 0, unroll=False)

    return k


def _scatter_make(n_pad, e, d):
    n_chunks = e // CH
    base_c, rem_c = n_chunks // NS, n_chunks % NS
    rows_per_tile = n_pad // NS
    r_chunks = rows_per_tile // CH
    mesh = plsc.VectorSubcoreMesh(core_axis_name="c", subcore_axis_name="s")

    @functools.partial(
        pl.kernel,
        out_type=jax.ShapeDtypeStruct((2 * n_pad, d), jnp.float32),
        mesh=mesh,
        scratch_types=[
            pltpu.VMEM((CH,), jnp.int32),
            pltpu.VMEM((CH, d), jnp.float32),
            pltpu.VMEM_SHARED((n_pad, d), jnp.float32),
        ],
    )
    def k(m_hbm, row_hbm, zmd_hbm, s_hbm, idx_v, mbuf, acc):
        cid = lax.axis_index("c")
        sid = lax.axis_index("s")

        pltpu.sync_copy(zmd_hbm, mbuf)
        row0 = sid * rows_per_tile
        for j in range(r_chunks):
            pltpu.sync_copy(mbuf, acc.at[pl.ds(row0 + j * CH, CH)])
        plsc.subcore_barrier()

        n_my = base_c + jnp.where(sid < rem_c, 1, 0)

        def body(i, _):
            off = (sid + i * NS) * CH
            pltpu.sync_copy(row_hbm.at[pl.ds(off, CH)], idx_v)
            pltpu.sync_copy(m_hbm.at[pl.ds(cid * e + off, CH)], mbuf)
            pltpu.sync_copy(mbuf, acc.at[idx_v], add=True)
            return 0

        lax.fori_loop(0, n_my, body, 0, unroll=False)
        plsc.subcore_barrier()

        for j in range(r_chunks):
            r = row0 + j * CH
            pltpu.sync_copy(acc.at[pl.ds(r, CH)], mbuf)
            pltpu.sync_copy(mbuf, s_hbm.at[pl.ds(cid * n_pad + r, CH)])

    return k


def _count_make(n_pad, e, d):
    half = e // CH // 2
    base_c, rem_c = half // NS, half % NS
    rows_per_tile = n_pad // NS
    r_chunks = rows_per_tile // CH
    mesh = plsc.VectorSubcoreMesh(core_axis_name="c", subcore_axis_name="s")

    @functools.partial(
        pl.kernel,
        out_type=jax.ShapeDtypeStruct((2 * n_pad, d), jnp.float32),
        mesh=mesh,
        scratch_types=[
            pltpu.VMEM((CH,), jnp.int32),
            pltpu.VMEM((CH, d), jnp.float32),
            pltpu.VMEM((CH, d), jnp.float32),
            pltpu.VMEM_SHARED((n_pad, d), jnp.float32),
        ],
    )
    def k(row_hbm, zmd_hbm, ones_hbm, c_hbm, idx_v, obuf, zbuf, cacc):
        cid = lax.axis_index("c")
        sid = lax.axis_index("s")

        pltpu.sync_copy(zmd_hbm, zbuf)
        pltpu.sync_copy(ones_hbm, obuf)
        row0 = sid * rows_per_tile
        for j in range(r_chunks):
            pltpu.sync_copy(zbuf, cacc.at[pl.ds(row0 + j * CH, CH)])
        plsc.subcore_barrier()

        n_my = base_c + jnp.where(sid < rem_c, 1, 0)

        def body(i, _):
            off = (cid * half + sid + i * NS) * CH
            pltpu.sync_copy(row_hbm.at[pl.ds(off, CH)], idx_v)
            pltpu.sync_copy(obuf, cacc.at[idx_v], add=True)
            return 0

        lax.fori_loop(0, n_my, body, 0, unroll=False)
        plsc.subcore_barrier()

        for j in range(r_chunks):
            r = row0 + j * CH
            pltpu.sync_copy(cacc.at[pl.ds(r, CH)], zbuf)
            pltpu.sync_copy(zbuf, c_hbm.at[pl.ds(cid * n_pad + r, CH)])

    return k


def _edge_tc_kernel(xg_ref, ea_ref, we1a, we1b, we1c, be1, we2, be2, we3, be3,
                    wn11a, wn11b, bn11, wn12, bn12, wn13, bn13,
                    e_ref, m_ref):
    xs = xg_ref[0]
    xd = xg_ref[1]
    ea = ea_ref[...]
    h1 = xs @ we1a[...] + xd @ we1b[...] + ea @ we1c[...] + be1[...]
    h1 = jnp.maximum(h1, 0.0)
    h2 = jnp.maximum(h1 @ we2[...] + be2[...], 0.0)
    e = h2 @ we3[...] + be3[...] + ea
    e_ref[...] = e
    g1 = jnp.maximum(xd @ wn11a[...] + e @ wn11b[...] + bn11[...], 0.0)
    g2 = jnp.maximum(g1 @ wn12[...] + bn12[...], 0.0)
    m = g2 @ wn13[...] + bn13[...]
    m_ref[0] = m[:, :128]
    m_ref[1] = m[:, 128:]


def _node_tc_kernel(x_ref, s0_ref, s1_ref, c0_ref, c1_ref, wn21a, wn21b,
                    wn21c, bn21, wn22, bn22, wn23, bn23, h_ref):
    x = x_ref[...]
    s0 = s0_ref[...]
    s1 = s1_ref[...]
    cnt = c0_ref[...][:, :1] + c1_ref[...][:, :1]
    inv = 1.0 / jnp.maximum(cnt, 1.0)
    aga = s0 * inv
    agb = s1 * inv
    h1 = jnp.maximum(x @ wn21a[...] + aga @ wn21b[...] + agb @ wn21c[...] + bn21[...], 0.0)
    h2 = jnp.maximum(h1 @ wn22[...] + bn22[...], 0.0)
    h_ref[...] = h2 @ wn23[...] + bn23[...] + x


def _full(w):
    return pl.BlockSpec(w.shape, lambda i: (0,) * w.ndim)


def kernel(x, edge_index, edge_attr, We1, be1, We2, be2, We3, be3, Wn11, bn11,
           Wn12, bn12, Wn13, bn13, Wn21, bn21, Wn22, bn22, Wn23, bn23):
    n, d = x.shape
    e = edge_index.shape[1]
    de = edge_attr.shape[1]
    row = edge_index[0].astype(jnp.int32)
    col = edge_index[1].astype(jnp.int32)

    xg = _gather_make(n, e, d)(x, row, col)

    We1a, We1b, We1c = We1[:d], We1[d:2 * d], We1[2 * d:]
    Wn11a, Wn11b = Wn11[:d], Wn11[d:]
    Wn21a, Wn21b, Wn21c = Wn21[:d], Wn21[d:d + 128], Wn21[d + 128:]

    be = 1280
    grid_e = e // be
    ew = [We1a, We1b, We1c, be1, We2, be2, We3, be3, Wn11a, Wn11b, bn11,
          Wn12, bn12, Wn13, bn13]
    e_out, m = pl.pallas_call(
        _edge_tc_kernel,
        grid=(grid_e,),
        in_specs=[
            pl.BlockSpec((2, be, d), lambda i: (0, i, 0)),
            pl.BlockSpec((be, de), lambda i: (i, 0)),
        ] + [_full(w) for w in ew],
        out_specs=[
            pl.BlockSpec((be, de), lambda i: (i, 0)),
            pl.BlockSpec((2, be, d), lambda i: (0, i, 0)),
        ],
        out_shape=[
            jax.ShapeDtypeStruct((e, de), jnp.float32),
            jax.ShapeDtypeStruct((2, e, d), jnp.float32),
        ],
        compiler_params=pltpu.CompilerParams(
            dimension_semantics=("arbitrary",)),
    )(xg, edge_attr, *ew)

    zmd = jnp.zeros((CH, d), jnp.float32)
    ones = jnp.ones((CH, d), jnp.float32)
    m_flat = m.reshape(2 * e, d)
    s = _scatter_make(_N_PAD, e, d)(m_flat, row, zmd)
    cnt = _count_make(_N_PAD, e, d)(row, zmd, ones)

    bn = 1024
    grid_n = (n + bn - 1) // bn
    s1_base = _N_PAD // bn
    nw = [Wn21a, Wn21b, Wn21c, bn21, Wn22, bn22, Wn23, bn23]
    h_out = pl.pallas_call(
        _node_tc_kernel,
        grid=(grid_n,),
        in_specs=[
            pl.BlockSpec((bn, d), lambda i: (i, 0)),
            pl.BlockSpec((bn, d), lambda i: (i, 0)),
            pl.BlockSpec((bn, d), lambda i: (s1_base + i, 0)),
            pl.BlockSpec((bn, d), lambda i: (i, 0)),
            pl.BlockSpec((bn, d), lambda i: (s1_base + i, 0)),
        ] + [_full(w) for w in nw],
        out_specs=pl.BlockSpec((bn, d), lambda i: (i, 0)),
        out_shape=jax.ShapeDtypeStruct((n, d), jnp.float32),
        compiler_params=pltpu.CompilerParams(
            dimension_semantics=("arbitrary",)),
    )(x, s, s, cnt, cnt, *nw)

    return h_out, e_out


# speedup vs baseline: 2.4613x; 1.0005x over previous
import functools

import jax
import jax.numpy as jnp
from jax import lax
from jax.experimental import pallas as pl
from jax.experimental.pallas import tpu as pltpu
from jax.experimental.pallas import tpu_sc as plsc

NC = 2
NS = 16
NW = NC * NS
CH = 128
D2 = 144

_N_PAD = 10240


def _gather_make(n, e, d):
    n_chunks = e // CH
    base_c, rem_c = n_chunks // NW, n_chunks % NW
    mesh = plsc.VectorSubcoreMesh(core_axis_name="c", subcore_axis_name="s")

    @functools.partial(
        pl.kernel,
        out_type=jax.ShapeDtypeStruct((2, e, d), jnp.float32),
        mesh=mesh,
        scratch_types=[
            pltpu.VMEM((CH,), jnp.int32),
            pltpu.VMEM((CH,), jnp.int32),
            pltpu.VMEM((CH, d), jnp.float32),
            pltpu.VMEM((CH, d), jnp.float32),
            pltpu.SemaphoreType.DMA,
            pltpu.SemaphoreType.DMA,
        ],
    )
    def k(x_hbm, row_hbm, col_hbm, out_hbm, idx_r, idx_c, buf_r, buf_c, sem_r, sem_c):
        cid = lax.axis_index("c")
        sid = lax.axis_index("s")
        wid = sid * NC + cid
        n_my = base_c + jnp.where(wid < rem_c, 1, 0)

        def body(i, _):
            off = (wid + i * NW) * CH
            pltpu.sync_copy(row_hbm.at[pl.ds(off, CH)], idx_r)
            pltpu.sync_copy(col_hbm.at[pl.ds(off, CH)], idx_c)
            cp_r = pltpu.async_copy(x_hbm.at[idx_r], buf_r, sem_r)
            cp_c = pltpu.async_copy(x_hbm.at[idx_c], buf_c, sem_c)
            cp_r.wait()
            cp_c.wait()
            pltpu.sync_copy(buf_r, out_hbm.at[0, pl.ds(off, CH)])
            pltpu.sync_copy(buf_c, out_hbm.at[1, pl.ds(off, CH)])
            return 0

        lax.fori_loop(0, n_my, body, 0, unroll=False)

    return k


def _scatter_make(n_pad, e, d):
    n_chunks = e // CH
    base_c, rem_c = n_chunks // NS, n_chunks % NS
    rows_per_tile = n_pad // NS
    r_chunks = rows_per_tile // CH
    mesh = plsc.VectorSubcoreMesh(core_axis_name="c", subcore_axis_name="s")

    @functools.partial(
        pl.kernel,
        out_type=jax.ShapeDtypeStruct((2 * n_pad, d), jnp.float32),
        mesh=mesh,
        scratch_types=[
            pltpu.VMEM((CH,), jnp.int32),
            pltpu.VMEM((CH, d), jnp.float32),
            pltpu.VMEM_SHARED((n_pad, d), jnp.float32),
        ],
    )
    def k(m_hbm, row_hbm, zmd_hbm, s_hbm, idx_v, mbuf, acc):
        cid = lax.axis_index("c")
        sid = lax.axis_index("s")

        pltpu.sync_copy(zmd_hbm, mbuf)
        row0 = sid * rows_per_tile
        for j in range(r_chunks):
            pltpu.sync_copy(mbuf, acc.at[pl.ds(row0 + j * CH, CH)])
        plsc.subcore_barrier()

        n_my = base_c + jnp.where(sid < rem_c, 1, 0)

        def body(i, _):
            off = (sid + i * NS) * CH
            pltpu.sync_copy(row_hbm.at[pl.ds(off, CH)], idx_v)
            pltpu.sync_copy(m_hbm.at[pl.ds(cid * e + off, CH)], mbuf)
            pltpu.sync_copy(mbuf, acc.at[idx_v], add=True)
            return 0

        lax.fori_loop(0, n_my, body, 0, unroll=False)
        plsc.subcore_barrier()

        for j in range(r_chunks):
            r = row0 + j * CH
            pltpu.sync_copy(acc.at[pl.ds(r, CH)], mbuf)
            pltpu.sync_copy(mbuf, s_hbm.at[pl.ds(cid * n_pad + r, CH)])

    return k


def _count_make(n_pad, e, d):
    half = e // CH // 2
    base_c, rem_c = half // NS, half % NS
    rows_per_tile = n_pad // NS
    r_chunks = rows_per_tile // CH
    mesh = plsc.VectorSubcoreMesh(core_axis_name="c", subcore_axis_name="s")

    @functools.partial(
        pl.kernel,
        out_type=jax.ShapeDtypeStruct((2 * n_pad, d), jnp.float32),
        mesh=mesh,
        scratch_types=[
            pltpu.VMEM((CH,), jnp.int32),
            pltpu.VMEM((CH, d), jnp.float32),
            pltpu.VMEM((CH, d), jnp.float32),
            pltpu.VMEM_SHARED((n_pad, d), jnp.float32),
        ],
    )
    def k(row_hbm, zmd_hbm, ones_hbm, c_hbm, idx_v, obuf, zbuf, cacc):
        cid = lax.axis_index("c")
        sid = lax.axis_index("s")

        pltpu.sync_copy(zmd_hbm, zbuf)
        pltpu.sync_copy(ones_hbm, obuf)
        row0 = sid * rows_per_tile
        for j in range(r_chunks):
            pltpu.sync_copy(zbuf, cacc.at[pl.ds(row0 + j * CH, CH)])
        plsc.subcore_barrier()

        n_my = base_c + jnp.where(sid < rem_c, 1, 0)

        def body(i, _):
            off = (cid * half + sid + i * NS) * CH
            pltpu.sync_copy(row_hbm.at[pl.ds(off, CH)], idx_v)
            pltpu.sync_copy(obuf, cacc.at[idx_v], add=True)
            return 0

        lax.fori_loop(0, n_my, body, 0, unroll=False)
        plsc.subcore_barrier()

        for j in range(r_chunks):
            r = row0 + j * CH
            pltpu.sync_copy(cacc.at[pl.ds(r, CH)], zbuf)
            pltpu.sync_copy(zbuf, c_hbm.at[pl.ds(cid * n_pad + r, CH)])

    return k


def _edge_tc_kernel(xg_ref, ea_ref, we1a, we1b, we1c, be1, we2, be2, we3, be3,
                    wn11a, wn11b, bn11, wn12, bn12, wn13, bn13,
                    e_ref, m_ref):
    bf = jnp.bfloat16
    f32 = jnp.float32

    def dot(a, b):
        return jax.lax.dot(a, b, preferred_element_type=f32)

    xs = xg_ref[0].astype(bf)
    xd = xg_ref[1].astype(bf)
    ea = ea_ref[...]
    eab = ea.astype(bf)
    h1 = dot(xs, we1a[...]) + dot(xd, we1b[...]) + dot(eab, we1c[...]) + be1[...]
    h1 = jnp.maximum(h1, 0.0).astype(bf)
    h2 = jnp.maximum(dot(h1, we2[...]) + be2[...], 0.0).astype(bf)
    e = dot(h2, we3[...]) + be3[...] + ea
    e_ref[...] = e
    g1 = dot(xd, wn11a[...]) + dot(e.astype(bf), wn11b[...]) + bn11[...]
    g1 = jnp.maximum(g1, 0.0).astype(bf)
    g2 = jnp.maximum(dot(g1, wn12[...]) + bn12[...], 0.0).astype(bf)
    m = dot(g2, wn13[...]) + bn13[...]
    m_ref[0] = m[:, :128]
    m_ref[1] = m[:, 128:]


def _node_tc_kernel(x_ref, s0_ref, s1_ref, c0_ref, c1_ref, wn21a, wn21b,
                    wn21c, bn21, wn22, bn22, wn23, bn23, h_ref):
    x = x_ref[...]
    s0 = s0_ref[...]
    s1 = s1_ref[...]
    cnt = c0_ref[...][:, :1] + c1_ref[...][:, :1]
    inv = 1.0 / jnp.maximum(cnt, 1.0)
    aga = s0 * inv
    agb = s1 * inv
    h1 = jnp.maximum(x @ wn21a[...] + aga @ wn21b[...] + agb @ wn21c[...] + bn21[...], 0.0)
    h2 = jnp.maximum(h1 @ wn22[...] + bn22[...], 0.0)
    h_ref[...] = h2 @ wn23[...] + bn23[...] + x


def _full(w):
    return pl.BlockSpec(w.shape, lambda i: (0,) * w.ndim)


def kernel(x, edge_index, edge_attr, We1, be1, We2, be2, We3, be3, Wn11, bn11,
           Wn12, bn12, Wn13, bn13, Wn21, bn21, Wn22, bn22, Wn23, bn23):
    n, d = x.shape
    e = edge_index.shape[1]
    de = edge_attr.shape[1]
    row = edge_index[0].astype(jnp.int32)
    col = edge_index[1].astype(jnp.int32)

    xg = _gather_make(n, e, d)(x, row, col)

    bf = jnp.bfloat16
    We1a, We1b, We1c = We1[:d].astype(bf), We1[d:2 * d].astype(bf), We1[2 * d:].astype(bf)
    Wn11a, Wn11b = Wn11[:d].astype(bf), Wn11[d:].astype(bf)
    Wn21a, Wn21b, Wn21c = Wn21[:d], Wn21[d:d + 128], Wn21[d + 128:]

    be = 1280
    grid_e = e // be
    ew = [We1a, We1b, We1c, be1, We2.astype(bf), be2, We3.astype(bf), be3,
          Wn11a, Wn11b, bn11, Wn12.astype(bf), bn12, Wn13.astype(bf), bn13]
    e_out, m = pl.pallas_call(
        _edge_tc_kernel,
        grid=(grid_e,),
        in_specs=[
            pl.BlockSpec((2, be, d), lambda i: (0, i, 0)),
            pl.BlockSpec((be, de), lambda i: (i, 0)),
        ] + [_full(w) for w in ew],
        out_specs=[
            pl.BlockSpec((be, de), lambda i: (i, 0)),
            pl.BlockSpec((2, be, d), lambda i: (0, i, 0)),
        ],
        out_shape=[
            jax.ShapeDtypeStruct((e, de), jnp.float32),
            jax.ShapeDtypeStruct((2, e, d), jnp.float32),
        ],
        compiler_params=pltpu.CompilerParams(
            dimension_semantics=("arbitrary",)),
    )(xg, edge_attr, *ew)

    zmd = jnp.zeros((CH, d), jnp.float32)
    ones = jnp.ones((CH, d), jnp.float32)
    m_flat = m.reshape(2 * e, d)
    s = _scatter_make(_N_PAD, e, d)(m_flat, row, zmd)
    cnt = _count_make(_N_PAD, e, d)(row, zmd, ones)

    bn = 1024
    grid_n = (n + bn - 1) // bn
    s1_base = _N_PAD // bn
    nw = [Wn21a, Wn21b, Wn21c, bn21, Wn22, bn22, Wn23, bn23]
    h_out = pl.pallas_call(
        _node_tc_kernel,
        grid=(grid_n,),
        in_specs=[
            pl.BlockSpec((bn, d), lambda i: (i, 0)),
            pl.BlockSpec((bn, d), lambda i: (i, 0)),
            pl.BlockSpec((bn, d), lambda i: (s1_base + i, 0)),
            pl.BlockSpec((bn, d), lambda i: (i, 0)),
            pl.BlockSpec((bn, d), lambda i: (s1_base + i, 0)),
        ] + [_full(w) for w in nw],
        out_specs=pl.BlockSpec((bn, d), lambda i: (i, 0)),
        out_shape=jax.ShapeDtypeStruct((n, d), jnp.float32),
        compiler_params=pltpu.CompilerParams(
            dimension_semantics=("arbitrary",)),
    )(x, s, s, cnt, cnt, *nw)

    return h_out, e_out


# speedup vs baseline: 3.0239x; 1.2286x over previous
import functools

import jax
import jax.numpy as jnp
from jax import lax
from jax.experimental import pallas as pl
from jax.experimental.pallas import tpu as pltpu
from jax.experimental.pallas import tpu_sc as plsc

NC = 2
NS = 16
NW = NC * NS
CH = 128
D2 = 144

_N_PAD = 10240


def _gather_make(n, e, d):
    n_chunks = e // CH
    base_c, rem_c = n_chunks // NW, n_chunks % NW
    mesh = plsc.VectorSubcoreMesh(core_axis_name="c", subcore_axis_name="s")

    @functools.partial(
        pl.kernel,
        out_type=jax.ShapeDtypeStruct((2, e, d), jnp.float32),
        mesh=mesh,
        scratch_types=[
            pltpu.VMEM((CH,), jnp.int32),
            pltpu.VMEM((CH,), jnp.int32),
            pltpu.VMEM((CH, d), jnp.float32),
            pltpu.VMEM((CH, d), jnp.float32),
            pltpu.SemaphoreType.DMA,
            pltpu.SemaphoreType.DMA,
        ],
    )
    def k(x_hbm, row_hbm, col_hbm, out_hbm, idx_r, idx_c, buf_r, buf_c, sem_r, sem_c):
        cid = lax.axis_index("c")
        sid = lax.axis_index("s")
        wid = sid * NC + cid
        n_my = base_c + jnp.where(wid < rem_c, 1, 0)

        def body(i, _):
            off = (wid + i * NW) * CH
            pltpu.sync_copy(row_hbm.at[pl.ds(off, CH)], idx_r)
            pltpu.sync_copy(col_hbm.at[pl.ds(off, CH)], idx_c)
            cp_r = pltpu.async_copy(x_hbm.at[idx_r], buf_r, sem_r)
            cp_c = pltpu.async_copy(x_hbm.at[idx_c], buf_c, sem_c)
            cp_r.wait()
            cp_c.wait()
            pltpu.sync_copy(buf_r, out_hbm.at[0, pl.ds(off, CH)])
            pltpu.sync_copy(buf_c, out_hbm.at[1, pl.ds(off, CH)])
            return 0

        lax.fori_loop(0, n_my, body, 0, unroll=False)

    return k


def _scatter_make(n_pad, e, d):
    n_chunks = e // CH
    base_c, rem_c = n_chunks // NS, n_chunks % NS
    rows_per_tile = n_pad // NS
    r_chunks = rows_per_tile // CH
    mesh = plsc.VectorSubcoreMesh(core_axis_name="c", subcore_axis_name="s")

    @functools.partial(
        pl.kernel,
        out_type=jax.ShapeDtypeStruct((2 * n_pad, d), jnp.float32),
        mesh=mesh,
        scratch_types=[
            pltpu.VMEM((CH,), jnp.int32),
            pltpu.VMEM((CH, d), jnp.float32),
            pltpu.VMEM_SHARED((n_pad, d), jnp.float32),
        ],
    )
    def k(m_hbm, row_hbm, zmd_hbm, s_hbm, idx_v, mbuf, acc):
        cid = lax.axis_index("c")
        sid = lax.axis_index("s")

        pltpu.sync_copy(zmd_hbm, mbuf)
        row0 = sid * rows_per_tile
        for j in range(r_chunks):
            pltpu.sync_copy(mbuf, acc.at[pl.ds(row0 + j * CH, CH)])
        plsc.subcore_barrier()

        n_my = base_c + jnp.where(sid < rem_c, 1, 0)

        def body(i, _):
            off = (sid + i * NS) * CH
            pltpu.sync_copy(row_hbm.at[pl.ds(off, CH)], idx_v)
            pltpu.sync_copy(m_hbm.at[pl.ds(cid * e + off, CH)], mbuf)
            pltpu.sync_copy(mbuf, acc.at[idx_v], add=True)
            return 0

        lax.fori_loop(0, n_my, body, 0, unroll=False)
        plsc.subcore_barrier()

        for j in range(r_chunks):
            r = row0 + j * CH
            pltpu.sync_copy(acc.at[pl.ds(r, CH)], mbuf)
            pltpu.sync_copy(mbuf, s_hbm.at[pl.ds(cid * n_pad + r, CH)])

    return k


def _count_make(n_pad, e, d):
    half = e // CH // 2
    base_c, rem_c = half // NS, half % NS
    rows_per_tile = n_pad // NS
    r_chunks = rows_per_tile // CH
    mesh = plsc.VectorSubcoreMesh(core_axis_name="c", subcore_axis_name="s")

    @functools.partial(
        pl.kernel,
        out_type=jax.ShapeDtypeStruct((2 * n_pad, d), jnp.float32),
        mesh=mesh,
        scratch_types=[
            pltpu.VMEM((CH,), jnp.int32),
            pltpu.VMEM((CH, d), jnp.float32),
            pltpu.VMEM((CH, d), jnp.float32),
            pltpu.VMEM_SHARED((n_pad, d), jnp.float32),
        ],
    )
    def k(row_hbm, zmd_hbm, ones_hbm, c_hbm, idx_v, obuf, zbuf, cacc):
        cid = lax.axis_index("c")
        sid = lax.axis_index("s")

        pltpu.sync_copy(zmd_hbm, zbuf)
        pltpu.sync_copy(ones_hbm, obuf)
        row0 = sid * rows_per_tile
        for j in range(r_chunks):
            pltpu.sync_copy(zbuf, cacc.at[pl.ds(row0 + j * CH, CH)])
        plsc.subcore_barrier()

        n_my = base_c + jnp.where(sid < rem_c, 1, 0)

        def body(i, _):
            off = (cid * half + sid + i * NS) * CH
            pltpu.sync_copy(row_hbm.at[pl.ds(off, CH)], idx_v)
            pltpu.sync_copy(obuf, cacc.at[idx_v], add=True)
            return 0

        lax.fori_loop(0, n_my, body, 0, unroll=False)
        plsc.subcore_barrier()

        for j in range(r_chunks):
            r = row0 + j * CH
            pltpu.sync_copy(cacc.at[pl.ds(r, CH)], zbuf)
            pltpu.sync_copy(zbuf, c_hbm.at[pl.ds(cid * n_pad + r, CH)])

    return k


def _edge_tc_kernel(xg_ref, ea_ref, we1a, we1b, we1c, be1, we2, be2, we3, be3,
                    wn11a, wn11b, bn11, wn12, bn12, wn13, bn13,
                    e_ref, m_ref):
    bf = jnp.bfloat16
    f32 = jnp.float32

    def dot(a, b):
        return jax.lax.dot(a, b, preferred_element_type=f32)

    xs = xg_ref[0].astype(bf)
    xd = xg_ref[1].astype(bf)
    ea = ea_ref[...]
    eab = ea.astype(bf)
    h1 = dot(xs, we1a[...]) + dot(xd, we1b[...]) + dot(eab, we1c[...]) + be1[...]
    h1 = jnp.maximum(h1, 0.0).astype(bf)
    h2 = jnp.maximum(dot(h1, we2[...]) + be2[...], 0.0).astype(bf)
    e = dot(h2, we3[...]) + be3[...] + ea
    e_ref[...] = e
    g1 = dot(xd, wn11a[...]) + dot(e.astype(bf), wn11b[...]) + bn11[...]
    g1 = jnp.maximum(g1, 0.0).astype(bf)
    g2 = jnp.maximum(dot(g1, wn12[...]) + bn12[...], 0.0).astype(bf)
    m = dot(g2, wn13[...]) + bn13[...]
    m_ref[0] = m[:, :128]
    m_ref[1] = m[:, 128:]


def _node_tc_kernel(x_ref, sa0_ref, sa1_ref, sb0_ref, sb1_ref, c0_ref, c1_ref,
                    wn21a, wn21b, wn21c, bn21, wn22, bn22, wn23, bn23, h_ref):
    x = x_ref[...]
    s0 = sa0_ref[...] + sb0_ref[...]
    s1 = sa1_ref[...] + sb1_ref[...]
    cnt = c0_ref[...][:, :1] + c1_ref[...][:, :1]
    inv = 1.0 / jnp.maximum(cnt, 1.0)
    aga = s0 * inv
    agb = s1 * inv
    h1 = jnp.maximum(x @ wn21a[...] + aga @ wn21b[...] + agb @ wn21c[...] + bn21[...], 0.0)
    h2 = jnp.maximum(h1 @ wn22[...] + bn22[...], 0.0)
    h_ref[...] = h2 @ wn23[...] + bn23[...] + x


def _full(w):
    return pl.BlockSpec(w.shape, lambda i: (0,) * w.ndim)


def kernel(x, edge_index, edge_attr, We1, be1, We2, be2, We3, be3, Wn11, bn11,
           Wn12, bn12, Wn13, bn13, Wn21, bn21, Wn22, bn22, Wn23, bn23):
    n, d = x.shape
    e = edge_index.shape[1]
    de = edge_attr.shape[1]
    row = edge_index[0].astype(jnp.int32)
    col = edge_index[1].astype(jnp.int32)

    bf = jnp.bfloat16
    We1a, We1b, We1c = We1[:d].astype(bf), We1[d:2 * d].astype(bf), We1[2 * d:].astype(bf)
    Wn11a, Wn11b = Wn11[:d].astype(bf), Wn11[d:].astype(bf)
    Wn21a, Wn21b, Wn21c = Wn21[:d], Wn21[d:d + 128], Wn21[d + 128:]

    eh = e // 2
    be = 1280
    grid_e = eh // be
    ew = [We1a, We1b, We1c, be1, We2.astype(bf), be2, We3.astype(bf), be3,
          Wn11a, Wn11b, bn11, Wn12.astype(bf), bn12, Wn13.astype(bf), bn13]

    def edge_stage(xg_h, ea_h):
        return pl.pallas_call(
            _edge_tc_kernel,
            grid=(grid_e,),
            in_specs=[
                pl.BlockSpec((2, be, d), lambda i: (0, i, 0)),
                pl.BlockSpec((be, de), lambda i: (i, 0)),
            ] + [_full(w) for w in ew],
            out_specs=[
                pl.BlockSpec((be, de), lambda i: (i, 0)),
                pl.BlockSpec((2, be, d), lambda i: (0, i, 0)),
            ],
            out_shape=[
                jax.ShapeDtypeStruct((eh, de), jnp.float32),
                jax.ShapeDtypeStruct((2, eh, d), jnp.float32),
            ],
            compiler_params=pltpu.CompilerParams(
                dimension_semantics=("arbitrary",)),
        )(xg_h, ea_h, *ew)

    zmd = jnp.zeros((CH, d), jnp.float32)
    ones = jnp.ones((CH, d), jnp.float32)

    row_a, row_b = row[:eh], row[eh:]
    col_a, col_b = col[:eh], col[eh:]
    gather = _gather_make(n, eh, d)
    scatter = _scatter_make(_N_PAD, eh, d)

    xg_a = gather(x, row_a, col_a)
    xg_b = gather(x, row_b, col_b)
    cnt = _count_make(_N_PAD, e, d)(row, zmd, ones)
    e_a, m_a = edge_stage(xg_a, edge_attr[:eh])
    s_a = scatter(m_a.reshape(2 * eh, d), row_a, zmd)
    e_b, m_b = edge_stage(xg_b, edge_attr[eh:])
    s_b = scatter(m_b.reshape(2 * eh, d), row_b, zmd)
    e_out = jnp.concatenate([e_a, e_b], axis=0)

    bn = 1024
    grid_n = (n + bn - 1) // bn
    s1_base = _N_PAD // bn
    nw = [Wn21a, Wn21b, Wn21c, bn21, Wn22, bn22, Wn23, bn23]
    h_out = pl.pallas_call(
        _node_tc_kernel,
        grid=(grid_n,),
        in_specs=[
            pl.BlockSpec((bn, d), lambda i: (i, 0)),
            pl.BlockSpec((bn, d), lambda i: (i, 0)),
            pl.BlockSpec((bn, d), lambda i: (s1_base + i, 0)),
            pl.BlockSpec((bn, d), lambda i: (i, 0)),
            pl.BlockSpec((bn, d), lambda i: (s1_base + i, 0)),
            pl.BlockSpec((bn, d), lambda i: (i, 0)),
            pl.BlockSpec((bn, d), lambda i: (s1_base + i, 0)),
        ] + [_full(w) for w in nw],
        out_specs=pl.BlockSpec((bn, d), lambda i: (i, 0)),
        out_shape=jax.ShapeDtypeStruct((n, d), jnp.float32),
        compiler_params=pltpu.CompilerParams(
            dimension_semantics=("arbitrary",)),
    )(x, s_a, s_a, s_b, s_b, cnt, cnt, *nw)

    return h_out, e_out


# speedup vs baseline: 3.3403x; 1.1046x over previous
import functools

import jax
import jax.numpy as jnp
from jax import lax
from jax.experimental import pallas as pl
from jax.experimental.pallas import tpu as pltpu
from jax.experimental.pallas import tpu_sc as plsc

NC = 2
NS = 16
NW = NC * NS
CH = 128
D2 = 144

_N_PAD = 10240


def _gather_make(n, e, d):
    n_chunks = e // CH
    base_c, rem_c = n_chunks // NW, n_chunks % NW
    mesh = plsc.VectorSubcoreMesh(core_axis_name="c", subcore_axis_name="s")

    @functools.partial(
        pl.kernel,
        out_type=jax.ShapeDtypeStruct((2, e, d), jnp.float32),
        mesh=mesh,
        scratch_types=[
            pltpu.VMEM((CH,), jnp.int32),
            pltpu.VMEM((CH,), jnp.int32),
            pltpu.VMEM((CH, d), jnp.float32),
            pltpu.VMEM((CH, d), jnp.float32),
            pltpu.SemaphoreType.DMA,
            pltpu.SemaphoreType.DMA,
        ],
    )
    def k(x_hbm, row_hbm, col_hbm, out_hbm, idx_r, idx_c, buf_r, buf_c, sem_r, sem_c):
        cid = lax.axis_index("c")
        sid = lax.axis_index("s")
        wid = sid * NC + cid
        n_my = base_c + jnp.where(wid < rem_c, 1, 0)

        def body(i, _):
            off = (wid + i * NW) * CH
            pltpu.sync_copy(row_hbm.at[pl.ds(off, CH)], idx_r)
            pltpu.sync_copy(col_hbm.at[pl.ds(off, CH)], idx_c)
            cp_r = pltpu.async_copy(x_hbm.at[idx_r], buf_r, sem_r)
            cp_c = pltpu.async_copy(x_hbm.at[idx_c], buf_c, sem_c)
            cp_r.wait()
            cp_c.wait()
            pltpu.sync_copy(buf_r, out_hbm.at[0, pl.ds(off, CH)])
            pltpu.sync_copy(buf_c, out_hbm.at[1, pl.ds(off, CH)])
            return 0

        lax.fori_loop(0, n_my, body, 0, unroll=False)

    return k


def _scatter_make(n_pad, e, d):
    n_chunks = e // CH
    base_c, rem_c = n_chunks // NS, n_chunks % NS
    rows_per_tile = n_pad // NS
    r_chunks = rows_per_tile // CH
    mesh = plsc.VectorSubcoreMesh(core_axis_name="c", subcore_axis_name="s")

    @functools.partial(
        pl.kernel,
        out_type=jax.ShapeDtypeStruct((2 * n_pad, d), jnp.float32),
        mesh=mesh,
        scratch_types=[
            pltpu.VMEM((CH,), jnp.int32),
            pltpu.VMEM((CH, d), jnp.float32),
            pltpu.VMEM_SHARED((n_pad, d), jnp.float32),
        ],
    )
    def k(m_hbm, row_hbm, zmd_hbm, s_hbm, idx_v, mbuf, acc):
        cid = lax.axis_index("c")
        sid = lax.axis_index("s")

        pltpu.sync_copy(zmd_hbm, mbuf)
        row0 = sid * rows_per_tile
        for j in range(r_chunks):
            pltpu.sync_copy(mbuf, acc.at[pl.ds(row0 + j * CH, CH)])
        plsc.subcore_barrier()

        n_my = base_c + jnp.where(sid < rem_c, 1, 0)

        def body(i, _):
            off = (sid + i * NS) * CH
            pltpu.sync_copy(row_hbm.at[pl.ds(off, CH)], idx_v)
            pltpu.sync_copy(m_hbm.at[pl.ds(cid * e + off, CH)], mbuf)
            pltpu.sync_copy(mbuf, acc.at[idx_v], add=True)
            return 0

        lax.fori_loop(0, n_my, body, 0, unroll=False)
        plsc.subcore_barrier()

        for j in range(r_chunks):
            r = row0 + j * CH
            pltpu.sync_copy(acc.at[pl.ds(r, CH)], mbuf)
            pltpu.sync_copy(mbuf, s_hbm.at[pl.ds(cid * n_pad + r, CH)])

    return k


def _count_make(n_pad, e, d):
    half = e // CH // 2
    base_c, rem_c = half // NS, half % NS
    rows_per_tile = n_pad // NS
    r_chunks = rows_per_tile // CH
    mesh = plsc.VectorSubcoreMesh(core_axis_name="c", subcore_axis_name="s")

    @functools.partial(
        pl.kernel,
        out_type=jax.ShapeDtypeStruct((2 * n_pad, d), jnp.float32),
        mesh=mesh,
        scratch_types=[
            pltpu.VMEM((CH,), jnp.int32),
            pltpu.VMEM((CH, d), jnp.float32),
            pltpu.VMEM((CH, d), jnp.float32),
            pltpu.VMEM_SHARED((n_pad, d), jnp.float32),
        ],
    )
    def k(row_hbm, zmd_hbm, ones_hbm, c_hbm, idx_v, obuf, zbuf, cacc):
        cid = lax.axis_index("c")
        sid = lax.axis_index("s")

        pltpu.sync_copy(zmd_hbm, zbuf)
        pltpu.sync_copy(ones_hbm, obuf)
        row0 = sid * rows_per_tile
        for j in range(r_chunks):
            pltpu.sync_copy(zbuf, cacc.at[pl.ds(row0 + j * CH, CH)])
        plsc.subcore_barrier()

        n_my = base_c + jnp.where(sid < rem_c, 1, 0)

        def body(i, _):
            off = (cid * half + sid + i * NS) * CH
            pltpu.sync_copy(row_hbm.at[pl.ds(off, CH)], idx_v)
            pltpu.sync_copy(obuf, cacc.at[idx_v], add=True)
            return 0

        lax.fori_loop(0, n_my, body, 0, unroll=False)
        plsc.subcore_barrier()

        for j in range(r_chunks):
            r = row0 + j * CH
            pltpu.sync_copy(cacc.at[pl.ds(r, CH)], zbuf)
            pltpu.sync_copy(zbuf, c_hbm.at[pl.ds(cid * n_pad + r, CH)])

    return k


def _edge_tc_kernel(xg_ref, ea_ref, we1a, we1b, we1c, be1, we2, be2, we3, be3,
                    wn11a, wn11b, bn11, wn12, bn12, wn13, bn13,
                    e_ref, m_ref):
    bf = jnp.bfloat16
    f32 = jnp.float32

    def dot(a, b):
        return jax.lax.dot(a, b, preferred_element_type=f32)

    xs = xg_ref[0].astype(bf)
    xd = xg_ref[1].astype(bf)
    ea = ea_ref[...]
    eab = ea.astype(bf)
    h1 = dot(xs, we1a[...]) + dot(xd, we1b[...]) + dot(eab, we1c[...]) + be1[...]
    h1 = jnp.maximum(h1, 0.0).astype(bf)
    h2 = jnp.maximum(dot(h1, we2[...]) + be2[...], 0.0).astype(bf)
    e = dot(h2, we3[...]) + be3[...] + ea
    e_ref[...] = e
    g1 = dot(xd, wn11a[...]) + dot(e.astype(bf), wn11b[...]) + bn11[...]
    g1 = jnp.maximum(g1, 0.0).astype(bf)
    g2 = jnp.maximum(dot(g1, wn12[...]) + bn12[...], 0.0).astype(bf)
    m = dot(g2, wn13[...]) + bn13[...]
    m_ref[0] = m[:, :128]
    m_ref[1] = m[:, 128:]


def _node_tc_kernel(x_ref, sa0_ref, sa1_ref, sb0_ref, sb1_ref,
                    sc0_ref, sc1_ref, sd0_ref, sd1_ref, c0_ref, c1_ref,
                    wn21a, wn21b, wn21c, bn21, wn22, bn22, wn23, bn23, h_ref):
    x = x_ref[...]
    s0 = sa0_ref[...] + sb0_ref[...] + sc0_ref[...] + sd0_ref[...]
    s1 = sa1_ref[...] + sb1_ref[...] + sc1_ref[...] + sd1_ref[...]
    cnt = c0_ref[...][:, :1] + c1_ref[...][:, :1]
    inv = 1.0 / jnp.maximum(cnt, 1.0)
    aga = s0 * inv
    agb = s1 * inv
    h1 = jnp.maximum(x @ wn21a[...] + aga @ wn21b[...] + agb @ wn21c[...] + bn21[...], 0.0)
    h2 = jnp.maximum(h1 @ wn22[...] + bn22[...], 0.0)
    h_ref[...] = h2 @ wn23[...] + bn23[...] + x


def _full(w):
    return pl.BlockSpec(w.shape, lambda i: (0,) * w.ndim)


def kernel(x, edge_index, edge_attr, We1, be1, We2, be2, We3, be3, Wn11, bn11,
           Wn12, bn12, Wn13, bn13, Wn21, bn21, Wn22, bn22, Wn23, bn23):
    n, d = x.shape
    e = edge_index.shape[1]
    de = edge_attr.shape[1]
    row = edge_index[0].astype(jnp.int32)
    col = edge_index[1].astype(jnp.int32)

    bf = jnp.bfloat16
    We1a, We1b, We1c = We1[:d].astype(bf), We1[d:2 * d].astype(bf), We1[2 * d:].astype(bf)
    Wn11a, Wn11b = Wn11[:d].astype(bf), Wn11[d:].astype(bf)
    Wn21a, Wn21b, Wn21c = Wn21[:d], Wn21[d:d + 128], Wn21[d + 128:]

    nsl = 4
    eh = e // nsl
    be = 1600
    grid_e = eh // be
    ew = [We1a, We1b, We1c, be1, We2.astype(bf), be2, We3.astype(bf), be3,
          Wn11a, Wn11b, bn11, Wn12.astype(bf), bn12, Wn13.astype(bf), bn13]

    def edge_stage(xg_h, ea_h):
        return pl.pallas_call(
            _edge_tc_kernel,
            grid=(grid_e,),
            in_specs=[
                pl.BlockSpec((2, be, d), lambda i: (0, i, 0)),
                pl.BlockSpec((be, de), lambda i: (i, 0)),
            ] + [_full(w) for w in ew],
            out_specs=[
                pl.BlockSpec((be, de), lambda i: (i, 0)),
                pl.BlockSpec((2, be, d), lambda i: (0, i, 0)),
            ],
            out_shape=[
                jax.ShapeDtypeStruct((eh, de), jnp.float32),
                jax.ShapeDtypeStruct((2, eh, d), jnp.float32),
            ],
            compiler_params=pltpu.CompilerParams(
                dimension_semantics=("arbitrary",)),
        )(xg_h, ea_h, *ew)

    zmd = jnp.zeros((CH, d), jnp.float32)
    ones = jnp.ones((CH, d), jnp.float32)

    gather = _gather_make(n, eh, d)
    scatter = _scatter_make(_N_PAD, eh, d)

    rows = [row[i * eh:(i + 1) * eh] for i in range(nsl)]
    cols = [col[i * eh:(i + 1) * eh] for i in range(nsl)]
    eas = [edge_attr[i * eh:(i + 1) * eh] for i in range(nsl)]

    xg0 = gather(x, rows[0], cols[0])
    xg1 = gather(x, rows[1], cols[1])
    cnt = _count_make(_N_PAD, e, d)(row, zmd, ones)
    e_parts, s_parts = [], []
    for i in range(nsl):
        e_i, m_i = edge_stage(xg0, eas[i])
        if i + 2 < nsl:
            xg0, xg1 = xg1, gather(x, rows[i + 2], cols[i + 2])
        else:
            xg0 = xg1
        s_parts.append(scatter(m_i.reshape(2 * eh, d), rows[i], zmd))
        e_parts.append(e_i)
    e_out = jnp.concatenate(e_parts, axis=0)

    bn = 1024
    grid_n = (n + bn - 1) // bn
    s1_base = _N_PAD // bn
    nw = [Wn21a, Wn21b, Wn21c, bn21, Wn22, bn22, Wn23, bn23]
    h_out = pl.pallas_call(
        _node_tc_kernel,
        grid=(grid_n,),
        in_specs=[pl.BlockSpec((bn, d), lambda i: (i, 0))] + [
            pl.BlockSpec((bn, d), ix)
            for _ in range(4)
            for ix in (lambda i: (i, 0), lambda i: (s1_base + i, 0))
        ] + [
            pl.BlockSpec((bn, d), lambda i: (i, 0)),
            pl.BlockSpec((bn, d), lambda i: (s1_base + i, 0)),
        ] + [_full(w) for w in nw],
        out_specs=pl.BlockSpec((bn, d), lambda i: (i, 0)),
        out_shape=jax.ShapeDtypeStruct((n, d), jnp.float32),
        compiler_params=pltpu.CompilerParams(
            dimension_semantics=("arbitrary",)),
    )(x, s_parts[0], s_parts[0], s_parts[1], s_parts[1],
      s_parts[2], s_parts[2], s_parts[3], s_parts[3], cnt, cnt, *nw)

    return h_out, e_out
